# baseline (speedup 1.0000x reference)
import functools

import jax
import jax.numpy as jnp
from jax.experimental import pallas as pl

_GAMMA = 0.2
_FW = 1.0 - _GAMMA

_BLK = 1000


def _fused_mlp_kernel(f_ref, s_ref, w1_ref, w2_ref, b12_ref, w3_ref, b3_ref,
                      out_ref):
    ebd = jnp.dot(f_ref[...], w1_ref[...], preferred_element_type=jnp.float32)
    ebd += jnp.dot(s_ref[...], w2_ref[...], preferred_element_type=jnp.float32)
    ebd = jnp.maximum(ebd + b12_ref[...], 0.0)
    out_ref[...] = (
        jnp.dot(ebd, w3_ref[...], preferred_element_type=jnp.float32)
        + b3_ref[...]
    )


@functools.partial(jax.jit, static_argnames=())
def _run(f, s, W1, b1, W2, b2, W3, b3):
    n, din = f.shape
    din3 = s.shape[1]
    h = W1.shape[1]
    c = W3.shape[1]

    w1s = _FW * W1
    w2s = _GAMMA * W2
    b12 = (_FW * b1 + _GAMMA * b2).reshape(1, h)
    b3r = b3.reshape(1, c)

    grid = (n // _BLK,)
    return pl.pallas_call(
        _fused_mlp_kernel,
        grid=grid,
        in_specs=[
            pl.BlockSpec((_BLK, din), lambda i: (i, 0)),
            pl.BlockSpec((_BLK, din3), lambda i: (i, 0)),
            pl.BlockSpec((din, h), lambda i: (0, 0)),
            pl.BlockSpec((din3, h), lambda i: (0, 0)),
            pl.BlockSpec((1, h), lambda i: (0, 0)),
            pl.BlockSpec((h, c), lambda i: (0, 0)),
            pl.BlockSpec((1, c), lambda i: (0, 0)),
        ],
        out_specs=pl.BlockSpec((_BLK, c), lambda i: (i, 0)),
        out_shape=jax.ShapeDtypeStruct((n, c), jnp.float32),
    )(f, s, w1s, w2s, b12, W3, b3r)


def kernel(f, s, adj_t, W1, b1, W2, b2, W3, b3):
    del adj_t
    return _run(f, s, W1, b1, W2, b2, W3, b3)


# speedup vs baseline: 1.1603x; 1.1603x over previous
import functools

import jax
import jax.numpy as jnp
from jax.experimental import pallas as pl
from jax.experimental.pallas import tpu as pltpu

_GAMMA = 0.2
_FW = 1.0 - _GAMMA

_BLK = 2000


def _fused_mlp_kernel(f_ref, s_ref, w12_ref, b12_ref, w3_ref, b3_ref,
                      out_ref):
    fs = jnp.concatenate((f_ref[...], s_ref[...]), axis=1)
    ebd = jnp.dot(fs, w12_ref[...], preferred_element_type=jnp.float32)
    ebd = jnp.maximum(ebd + b12_ref[...], 0.0)
    out_ref[...] = (
        jnp.dot(ebd, w3_ref[...], preferred_element_type=jnp.float32)
        + b3_ref[...]
    )


@functools.partial(jax.jit, static_argnames=())
def _run(f, s, W1, b1, W2, b2, W3, b3):
    n, din = f.shape
    din3 = s.shape[1]
    h = W1.shape[1]
    c = W3.shape[1]

    w12 = jnp.concatenate((_FW * W1, _GAMMA * W2), axis=0)
    b12 = (_FW * b1 + _GAMMA * b2).reshape(1, h)
    b3r = b3.reshape(1, c)

    grid = (n // _BLK,)
    return pl.pallas_call(
        _fused_mlp_kernel,
        grid=grid,
        in_specs=[
            pl.BlockSpec((_BLK, din), lambda i: (i, 0)),
            pl.BlockSpec((_BLK, din3), lambda i: (i, 0)),
            pl.BlockSpec((din + din3, h), lambda i: (0, 0)),
            pl.BlockSpec((1, h), lambda i: (0, 0)),
            pl.BlockSpec((h, c), lambda i: (0, 0)),
            pl.BlockSpec((1, c), lambda i: (0, 0)),
        ],
        out_specs=pl.BlockSpec((_BLK, c), lambda i: (i, 0)),
        out_shape=jax.ShapeDtypeStruct((n, c), jnp.float32),
        compiler_params=pltpu.CompilerParams(
            dimension_semantics=("parallel",),
        ),
    )(f, s, w12, b12, W3, b3r)


def kernel(f, s, adj_t, W1, b1, W2, b2, W3, b3):
    del adj_t
    return _run(f, s, W1, b1, W2, b2, W3, b3)


# speedup vs baseline: 1.2869x; 1.1092x over previous
import functools

import jax
import jax.numpy as jnp
from jax.experimental import pallas as pl
from jax.experimental.pallas import tpu as pltpu

_GAMMA = 0.2
_FW = 1.0 - _GAMMA

_BLK = 5000


def _fused_mlp_kernel(f_ref, s_ref, w12_ref, b12_ref, w3_ref, b3_ref,
                      out_ref):
    fs = jnp.concatenate((f_ref[...], s_ref[...]), axis=1)
    ebd = jnp.dot(fs, w12_ref[...], preferred_element_type=jnp.float32)
    ebd = jnp.maximum(ebd + b12_ref[...], 0.0)
    out_ref[...] = (
        jnp.dot(ebd, w3_ref[...], preferred_element_type=jnp.float32)
        + b3_ref[...]
    )


@functools.partial(jax.jit, static_argnames=())
def _run(f, s, W1, b1, W2, b2, W3, b3):
    n, din = f.shape
    din3 = s.shape[1]
    h = W1.shape[1]
    c = W3.shape[1]

    w12 = jnp.concatenate((_FW * W1, _GAMMA * W2), axis=0)
    b12 = (_FW * b1 + _GAMMA * b2).reshape(1, h)
    b3r = b3.reshape(1, c)

    grid = (n // _BLK,)
    return pl.pallas_call(
        _fused_mlp_kernel,
        grid=grid,
        in_specs=[
            pl.BlockSpec((_BLK, din), lambda i: (i, 0)),
            pl.BlockSpec((_BLK, din3), lambda i: (i, 0)),
            pl.BlockSpec((din + din3, h), lambda i: (0, 0)),
            pl.BlockSpec((1, h), lambda i: (0, 0)),
            pl.BlockSpec((h, c), lambda i: (0, 0)),
            pl.BlockSpec((1, c), lambda i: (0, 0)),
        ],
        out_specs=pl.BlockSpec((_BLK, c), lambda i: (i, 0)),
        out_shape=jax.ShapeDtypeStruct((n, c), jnp.float32),
        compiler_params=pltpu.CompilerParams(
            dimension_semantics=("parallel",),
        ),
    )(f, s, w12, b12, W3, b3r)


def kernel(f, s, adj_t, W1, b1, W2, b2, W3, b3):
    del adj_t
    return _run(f, s, W1, b1, W2, b2, W3, b3)


# speedup vs baseline: 1.2912x; 1.0033x over previous
import functools

import jax
import jax.numpy as jnp
from jax.experimental import pallas as pl
from jax.experimental.pallas import tpu as pltpu

_GAMMA = 0.2
_FW = 1.0 - _GAMMA

_BLK = 5000


def _fused_mlp_kernel(f_ref, s_ref, w12_ref, b12_ref, w3_ref, b3_ref,
                      out_ref):
    fs = jnp.concatenate((f_ref[...], s_ref[...]), axis=1)
    ebd = jnp.dot(fs.astype(jnp.bfloat16), w12_ref[...],
                  preferred_element_type=jnp.float32)
    ebd = jnp.maximum(ebd + b12_ref[...], 0.0)
    out_ref[...] = (
        jnp.dot(ebd.astype(jnp.bfloat16), w3_ref[...],
                preferred_element_type=jnp.float32)
        + b3_ref[...]
    )


@functools.partial(jax.jit, static_argnames=())
def _run(f, s, W1, b1, W2, b2, W3, b3):
    n, din = f.shape
    din3 = s.shape[1]
    h = W1.shape[1]
    c = W3.shape[1]

    w12 = jnp.concatenate((_FW * W1, _GAMMA * W2), axis=0).astype(jnp.bfloat16)
    b12 = (_FW * b1 + _GAMMA * b2).reshape(1, h)
    b3r = b3.reshape(1, c)
    w3b = W3.astype(jnp.bfloat16)

    grid = (n // _BLK,)
    return pl.pallas_call(
        _fused_mlp_kernel,
        grid=grid,
        in_specs=[
            pl.BlockSpec((_BLK, din), lambda i: (i, 0)),
            pl.BlockSpec((_BLK, din3), lambda i: (i, 0)),
            pl.BlockSpec((din + din3, h), lambda i: (0, 0)),
            pl.BlockSpec((1, h), lambda i: (0, 0)),
            pl.BlockSpec((h, c), lambda i: (0, 0)),
            pl.BlockSpec((1, c), lambda i: (0, 0)),
        ],
        out_specs=pl.BlockSpec((_BLK, c), lambda i: (i, 0)),
        out_shape=jax.ShapeDtypeStruct((n, c), jnp.float32),
        compiler_params=pltpu.CompilerParams(
            dimension_semantics=("parallel",),
        ),
    )(f, s, w12, b12, w3b, b3r)


def kernel(f, s, adj_t, W1, b1, W2, b2, W3, b3):
    del adj_t
    return _run(f, s, W1, b1, W2, b2, W3, b3)
